# baseline (speedup 1.0000x reference)
import functools

import jax
import jax.numpy as jnp
from jax import lax
from jax.experimental import pallas as pl
from jax.experimental.pallas import tpu as pltpu
from jax.experimental.pallas import tpu_sc as plsc

N = 10000
E = 320000
HID = 128
BETA = 0.05
NC, NS = 2, 16
NW = NC * NS
JPT = 80
EPT = JPT * 128
EP = EPT * NW
EJ = EP // 128
NP = 10240
NG = 10016
RPT = NP // NS

_f32 = jnp.float32
_i32 = jnp.int32

_mesh = plsc.VectorSubcoreMesh(core_axis_name="c", subcore_axis_name="s")


def _zvec():
    return jnp.zeros((16,), _f32)


def _sc_logits_body(src_h, dst_h, et_h, si_h, sj_h, srm_h, e_h, ssum_h,
                    src_v, dst_v, et_v, e_v, si_v, sj_v, srm_v, zbuf,
                    ssum_sh):
    c = lax.axis_index("c")
    s = lax.axis_index("s")
    wid = c * NS + s
    base = wid * JPT

    def zb(k, _):
        zbuf[pl.ds(k * 16, 16)] = _zvec()
        return 0
    lax.fori_loop(0, RPT // 16, zb, 0)
    pltpu.sync_copy(zbuf, ssum_sh.at[pl.ds(s * RPT, RPT)])

    pltpu.sync_copy(src_h.at[pl.ds(base, JPT)], src_v)
    pltpu.sync_copy(dst_h.at[pl.ds(base, JPT)], dst_v)
    pltpu.sync_copy(et_h.at[pl.ds(base, JPT)], et_v)
    pltpu.sync_copy(si_h, si_v)
    pltpu.sync_copy(sj_h, sj_v)
    pltpu.sync_copy(srm_h, srm_v)
    plsc.subcore_barrier()

    def mxi(i, cm):
        return jnp.maximum(cm, si_v[pl.ds(i * 16, 16)])

    def mxj(i, cm):
        return jnp.maximum(cm, sj_v[pl.ds(i * 16, 16)])

    def _lanes_max(v):
        m = v[0]
        for i in range(1, 16):
            m = jnp.maximum(m, v[i])
        return m

    neg = jnp.full((16,), -1e30, _f32)
    msi = _lanes_max(lax.fori_loop(0, NP // 16, mxi, neg))
    msj = _lanes_max(lax.fori_loop(0, NP // 16, mxj, neg))
    sr0 = srm_v[0, pl.ds(0, 16)]
    sr1 = srm_v[1, pl.ds(0, 16)]
    sr2 = srm_v[2, pl.ds(0, 16)]
    sr3 = srm_v[3, pl.ds(0, 16)]
    msr = jnp.maximum(jnp.maximum(sr0, sr1), jnp.maximum(sr2, sr3))[0]
    mb = msi + msj + msr
    m_bound = jnp.where(mb >= 0, mb, 0.2 * mb)

    zero16 = jnp.zeros((16,), _i32)

    def per_chunk(j, _):
        for i in range(8):
            sl = pl.ds(i * 16, 16)
            dv = dst_v[j, sl]
            sv = src_v[j, sl]
            tv = et_v[j, sl]
            vsi = plsc.load_gather(si_v, [dv])
            vsj = plsc.load_gather(sj_v, [sv])
            vsr = plsc.load_gather(srm_v, [tv, zero16])
            lg = vsi + vsj + vsr
            lg = jnp.where(lg >= 0, lg, 0.2 * lg)
            e_v[j, sl] = jnp.exp(lg - m_bound)
        pltpu.sync_copy(e_v.at[j], ssum_sh.at[dst_v.at[j]], add=True)
        return 0
    lax.fori_loop(0, JPT, per_chunk, 0)

    pltpu.sync_copy(e_v, e_h.at[pl.ds(base, JPT)])
    plsc.subcore_barrier()
    pltpu.sync_copy(ssum_sh.at[pl.ds(s * RPT, RPT)],
                    ssum_h.at[c, pl.ds(s * RPT, RPT)])


_SC_PARAMS = pltpu.CompilerParams(needs_layout_passes=False)

_sc_logits = pl.kernel(
    _sc_logits_body,
    compiler_params=_SC_PARAMS,
    out_type=(
        jax.ShapeDtypeStruct((EJ, 128), _f32),
        jax.ShapeDtypeStruct((NC, NP), _f32),
    ),
    mesh=_mesh,
    scratch_types=[
        pltpu.VMEM((JPT, 128), _i32),
        pltpu.VMEM((JPT, 128), _i32),
        pltpu.VMEM((JPT, 128), _i32),
        pltpu.VMEM((JPT, 128), _f32),
        pltpu.VMEM((NP,), _f32),
        pltpu.VMEM((NP,), _f32),
        pltpu.VMEM((8, 128), _f32),
        pltpu.VMEM((RPT,), _f32),
        pltpu.VMEM_SHARED((NP,), _f32),
    ],
)


STRIP = 8
NSTRIP = JPT // STRIP


def _make_sc_agg(with_pre):
    def body(*args):
        if with_pre:
            (src_h, dst_h, e_h, r_h, g_h, pre_h, agg_h,
             src_s, dst_s, e_s, pre_s, alpha_s, r_v, rows_a, rows_b,
             agg_sh, sema, semb) = args
        else:
            (src_h, dst_h, e_h, r_h, g_h, alpha_h, agg_h,
             src_s, dst_s, e_s, alpha_s, r_v, rows_a, rows_b,
             agg_sh, sema, semb) = args
        c = lax.axis_index("c")
        s = lax.axis_index("s")
        wid = (1 - c) * NS + s
        base = wid * JPT

        pltpu.sync_copy(r_h, r_v)

        def zr(r, _):
            for f in range(8):
                rows_a[r, pl.ds(f * 16, 16)] = _zvec()
            return 0
        lax.fori_loop(0, 128, zr, 0)
        for k in range(RPT // 128):
            pltpu.sync_copy(rows_a, agg_sh.at[pl.ds(s * RPT + k * 128, 128)])
        plsc.subcore_barrier()

        bufs = (rows_a, rows_b)
        sems = (sema, semb)

        def per_strip(t, _):
            row0 = pl.multiple_of(base + t * STRIP, 8)
            pltpu.sync_copy(src_h.at[pl.ds(row0, STRIP)], src_s)
            pltpu.sync_copy(dst_h.at[pl.ds(row0, STRIP)], dst_s)
            pltpu.sync_copy(e_h.at[pl.ds(row0, STRIP)], e_s)
            if with_pre:
                pltpu.sync_copy(pre_h.at[pl.ds(row0, STRIP)], pre_s)

            cp = pltpu.async_copy(g_h.at[src_s.at[0]], rows_a, sema)
            for j in range(STRIP):
                buf = bufs[j % 2]
                if j + 1 < STRIP:
                    cp_next = pltpu.async_copy(
                        g_h.at[src_s.at[j + 1]], bufs[(j + 1) % 2],
                        sems[(j + 1) % 2])
                for i in range(8):
                    sl = pl.ds(i * 16, 16)
                    ev = e_s[j, sl]
                    dv = dst_s[j, sl]
                    hi = lax.shift_right_logical(dv, 7)
                    lo = jnp.bitwise_and(dv, 127)
                    rv = plsc.load_gather(r_v, [hi, lo])
                    av = ev * rv
                    if with_pre:
                        av = av * (1.0 - BETA) + pre_s[j, sl] * BETA
                    alpha_s[j, sl] = av
                cp.wait()

                def rbody(ri, _):
                    av = alpha_s[j, pl.ds(ri * 16, 16)]
                    for r0 in range(16):
                        sc = av[r0]
                        row = ri * 16 + r0
                        for f in range(8):
                            fl = pl.ds(f * 16, 16)
                            buf[row, fl] = buf[row, fl] * sc
                    return 0
                lax.fori_loop(0, 8, rbody, 0)
                pltpu.sync_copy(buf, agg_sh.at[dst_s.at[j]], add=True)
                if j + 1 < STRIP:
                    cp = cp_next
            if not with_pre:
                pltpu.sync_copy(alpha_s, alpha_h.at[pl.ds(row0, STRIP)])
            return 0
        lax.fori_loop(0, NSTRIP, per_strip, 0)

        plsc.subcore_barrier()
        pltpu.sync_copy(agg_sh.at[pl.ds(s * RPT, RPT)],
                        agg_h.at[c, pl.ds(s * RPT, RPT)])

    outs = [jax.ShapeDtypeStruct((NC, NP, 128), _f32)]
    if not with_pre:
        outs = [jax.ShapeDtypeStruct((EJ, 128), _f32)] + outs
    scratch = [
        pltpu.VMEM((STRIP, 128), _i32),
        pltpu.VMEM((STRIP, 128), _i32),
        pltpu.VMEM((STRIP, 128), _f32),
    ]
    if with_pre:
        scratch.append(pltpu.VMEM((STRIP, 128), _f32))
    scratch += [
        pltpu.VMEM((STRIP, 128), _f32),
        pltpu.VMEM((NP // 128, 128), _f32),
        pltpu.VMEM((128, 128), _f32),
        pltpu.VMEM((128, 128), _f32),
        pltpu.VMEM_SHARED((NP, 128), _f32),
        pltpu.SemaphoreType.DMA,
        pltpu.SemaphoreType.DMA,
    ]
    return pl.kernel(body, out_type=tuple(outs), mesh=_mesh,
                     compiler_params=_SC_PARAMS, scratch_types=scratch)


_sc_agg1 = _make_sc_agg(with_pre=False)
_sc_agg2 = _make_sc_agg(with_pre=True)


def _leaky(x, slope):
    return jnp.where(x >= 0, x, slope * x)


def _elu(x):
    return jnp.where(x > 0, x, jnp.exp(jnp.minimum(x, 0.0)) - 1.0)


BLK = 1000


def _tc1_body(f_ref, w1_ref, b1_ref, wl_ref, wres_ref, a2_ref, rel_ref,
              wr_ref, ar_ref, g_ref, sij_ref, xres_ref, srm_ref):
    h1 = _leaky(f_ref[...] @ w1_ref[...] + b1_ref[...], 0.01)
    g = h1 @ wl_ref[...]
    g_ref[...] = g
    sij_ref[...] = g @ a2_ref[...]
    xres_ref[...] = h1 @ wres_ref[...]
    srm_ref[...] = jnp.broadcast_to(
        (rel_ref[...] @ wr_ref[...]) @ ar_ref[...], (8, 128))


def _tc1(feature, W1, b1r, Wl, Wres, A2, relp, Wrp, AR):
    return pl.pallas_call(
        _tc1_body,
        grid=(N // BLK,),
        in_specs=[
            pl.BlockSpec((BLK, HID), lambda i: (i, 0)),
            pl.BlockSpec((HID, HID), lambda i: (0, 0)),
            pl.BlockSpec((1, HID), lambda i: (0, 0)),
            pl.BlockSpec((HID, HID), lambda i: (0, 0)),
            pl.BlockSpec((HID, HID), lambda i: (0, 0)),
            pl.BlockSpec((HID, 2), lambda i: (0, 0)),
            pl.BlockSpec((8, HID), lambda i: (0, 0)),
            pl.BlockSpec((HID, HID), lambda i: (0, 0)),
            pl.BlockSpec((HID, 1), lambda i: (0, 0)),
        ],
        out_specs=[
            pl.BlockSpec((BLK, HID), lambda i: (i, 0)),
            pl.BlockSpec((BLK, 2), lambda i: (i, 0)),
            pl.BlockSpec((BLK, HID), lambda i: (i, 0)),
            pl.BlockSpec((8, HID), lambda i: (0, 0)),
        ],
        out_shape=[
            jax.ShapeDtypeStruct((N, HID), _f32),
            jax.ShapeDtypeStruct((N, 2), _f32),
            jax.ShapeDtypeStruct((N, HID), _f32),
            jax.ShapeDtypeStruct((8, HID), _f32),
        ],
    )(feature, W1, b1r, Wl, Wres, A2, relp, Wrp, AR)


def _rsum_body(s_ref, r_ref):
    r_ref[...] = 1.0 / (s_ref[0] + s_ref[1] + 1e-16)


def _rsum(ssum):
    return pl.pallas_call(
        _rsum_body,
        out_shape=jax.ShapeDtypeStruct((NP // 128, 128), _f32),
    )(ssum.reshape(NC, NP // 128, 128))


def _tc2_body(agg_ref, xres_ref, bres_ref, wl_ref, wres_ref,
              a2_ref, g_ref, sij_ref, xres2_ref):
    x2 = _elu(agg_ref[0] + agg_ref[1] + xres_ref[...] + bres_ref[...])
    g = x2 @ wl_ref[...]
    g_ref[...] = g
    sij_ref[...] = g @ a2_ref[...]
    xres2_ref[...] = x2 @ wres_ref[...]


def _tc2(agg, xres1, bresr, Wl, Wres, A2):
    return pl.pallas_call(
        _tc2_body,
        grid=(N // BLK,),
        in_specs=[
            pl.BlockSpec((NC, BLK, HID), lambda i: (0, i, 0)),
            pl.BlockSpec((BLK, HID), lambda i: (i, 0)),
            pl.BlockSpec((1, HID), lambda i: (0, 0)),
            pl.BlockSpec((HID, HID), lambda i: (0, 0)),
            pl.BlockSpec((HID, HID), lambda i: (0, 0)),
            pl.BlockSpec((HID, 2), lambda i: (0, 0)),
        ],
        out_specs=[
            pl.BlockSpec((BLK, HID), lambda i: (i, 0)),
            pl.BlockSpec((BLK, 2), lambda i: (i, 0)),
            pl.BlockSpec((BLK, HID), lambda i: (i, 0)),
        ],
        out_shape=[
            jax.ShapeDtypeStruct((N, HID), _f32),
            jax.ShapeDtypeStruct((N, 2), _f32),
            jax.ShapeDtypeStruct((N, HID), _f32),
        ],
    )(agg, xres1, bresr, Wl, Wres, A2)


def _tc3_body(agg_ref, xres_ref, bres_ref, wo1_ref, bo1_ref,
              wo2_ref, bo2_ref, o_ref):
    x3 = _elu(agg_ref[0] + agg_ref[1] + xres_ref[...] + bres_ref[...])
    t = _leaky(x3 @ wo1_ref[...] + bo1_ref[...], 0.01)
    o_ref[...] = t @ wo2_ref[...] + bo2_ref[...]


def _tc3(agg, xres2, bresr, Wo1, bo1r, Wo2p, bo2r):
    return pl.pallas_call(
        _tc3_body,
        grid=(N // BLK,),
        in_specs=[
            pl.BlockSpec((NC, BLK, HID), lambda i: (0, i, 0)),
            pl.BlockSpec((BLK, HID), lambda i: (i, 0)),
            pl.BlockSpec((1, HID), lambda i: (0, 0)),
            pl.BlockSpec((HID, 64), lambda i: (0, 0)),
            pl.BlockSpec((1, 64), lambda i: (0, 0)),
            pl.BlockSpec((64, HID), lambda i: (0, 0)),
            pl.BlockSpec((1, HID), lambda i: (0, 0)),
        ],
        out_specs=pl.BlockSpec((BLK, HID), lambda i: (i, 0)),
        out_shape=jax.ShapeDtypeStruct((N, HID), _f32),
    )(agg, xres2, bresr, Wo1, bo1r, Wo2p, bo2r)


def kernel(feature, edge_index, edge_type, W1, b1, Wl, Wr, a, Wres, bres,
           rel_emb, Wo1, bo1, Wo2, bo2):
    src = edge_index[0].astype(_i32)
    dst = edge_index[1].astype(_i32)
    et = edge_type.astype(_i32)

    pad = EP - E
    src2 = jnp.concatenate([src, jnp.full((pad,), N, _i32)]).reshape(EJ, 128)
    dst2 = jnp.concatenate([dst, jnp.full((pad,), N, _i32)]).reshape(EJ, 128)
    et2 = jnp.concatenate([et, jnp.zeros((pad,), _i32)]).reshape(EJ, 128)

    b1r = b1.reshape(1, HID)
    bresr = bres.reshape(1, HID)
    bo1r = bo1.reshape(1, 64)
    A2 = jnp.concatenate([a[0:HID], a[HID:2 * HID]], axis=1)
    AR = a[2 * HID:3 * HID]
    relp = jnp.zeros((8, HID), _f32).at[:4, :100].set(rel_emb)
    Wrp = jnp.zeros((HID, HID), _f32).at[:100].set(Wr)
    Wo2p = jnp.zeros((64, HID), _f32).at[:, :2].set(Wo2)
    bo2r = jnp.zeros((1, HID), _f32).at[0, :2].set(bo2)

    g1, sij1, xres1, srm = _tc1(feature, W1, b1r, Wl, Wres, A2, relp, Wrp, AR)
    si1 = jnp.pad(sij1[:, 0], (0, NP - N))
    sj1 = jnp.pad(sij1[:, 1], (0, NP - N))
    g1p = jnp.pad(g1, ((0, NG - N), (0, 0)))

    e1, ssum1 = _sc_logits(src2, dst2, et2, si1, sj1, srm)
    alpha1, agg1 = _sc_agg1(src2, dst2, e1, _rsum(ssum1), g1p)

    g2, sij2, xres2 = _tc2(agg1, xres1, bresr, Wl, Wres, A2)
    si2 = jnp.pad(sij2[:, 0], (0, NP - N))
    sj2 = jnp.pad(sij2[:, 1], (0, NP - N))
    g2p = jnp.pad(g2, ((0, NG - N), (0, 0)))

    e2, ssum2 = _sc_logits(src2, dst2, et2, si2, sj2, srm)
    (agg2,) = _sc_agg2(src2, dst2, e2, _rsum(ssum2), g2p, alpha1)

    out = _tc3(agg2, xres2, bresr, Wo1, bo1r, Wo2p, bo2r)
    return out[:, :2]


# speedup vs baseline: 2.4627x; 2.4627x over previous
import functools

import jax
import jax.numpy as jnp
from jax import lax
from jax.experimental import pallas as pl
from jax.experimental.pallas import tpu as pltpu
from jax.experimental.pallas import tpu_sc as plsc

N = 10000
E = 320000
HID = 128
BETA = 0.05
NC, NS = 2, 16
NW = NC * NS
JPT = 80
EPT = JPT * 128
EP = EPT * NW
EJ = EP // 128
NP = 10240
NG = 10016
RPT = NP // NS

_f32 = jnp.float32
_i32 = jnp.int32

_mesh = plsc.VectorSubcoreMesh(core_axis_name="c", subcore_axis_name="s")


def _zvec():
    return jnp.zeros((16,), _f32)


def _sc_logits_body(src_h, dst_h, et_h, si_h, sj_h, srm_h, e_h, ssum_h,
                    src_v, dst_v, et_v, e_v, si_v, sj_v, srm_v, zbuf,
                    ssum_sh):
    c = lax.axis_index("c")
    s = lax.axis_index("s")
    wid = c * NS + s
    base = wid * JPT

    def zb(k, _):
        zbuf[pl.ds(k * 16, 16)] = _zvec()
        return 0
    lax.fori_loop(0, RPT // 16, zb, 0)
    pltpu.sync_copy(zbuf, ssum_sh.at[pl.ds(s * RPT, RPT)])

    pltpu.sync_copy(src_h.at[pl.ds(base, JPT)], src_v)
    pltpu.sync_copy(dst_h.at[pl.ds(base, JPT)], dst_v)
    pltpu.sync_copy(et_h.at[pl.ds(base, JPT)], et_v)
    pltpu.sync_copy(si_h, si_v)
    pltpu.sync_copy(sj_h, sj_v)
    pltpu.sync_copy(srm_h, srm_v)
    plsc.subcore_barrier()

    def mxi(i, cm):
        return jnp.maximum(cm, si_v[pl.ds(i * 16, 16)])

    def mxj(i, cm):
        return jnp.maximum(cm, sj_v[pl.ds(i * 16, 16)])

    def _lanes_max(v):
        m = v[0]
        for i in range(1, 16):
            m = jnp.maximum(m, v[i])
        return m

    neg = jnp.full((16,), -1e30, _f32)
    msi = _lanes_max(lax.fori_loop(0, NP // 16, mxi, neg))
    msj = _lanes_max(lax.fori_loop(0, NP // 16, mxj, neg))
    sr0 = srm_v[0, pl.ds(0, 16)]
    sr1 = srm_v[1, pl.ds(0, 16)]
    sr2 = srm_v[2, pl.ds(0, 16)]
    sr3 = srm_v[3, pl.ds(0, 16)]
    msr = jnp.maximum(jnp.maximum(sr0, sr1), jnp.maximum(sr2, sr3))[0]
    mb = msi + msj + msr
    m_bound = jnp.where(mb >= 0, mb, 0.2 * mb)

    zero16 = jnp.zeros((16,), _i32)

    def per_chunk(j, _):
        for i in range(8):
            sl = pl.ds(i * 16, 16)
            dv = dst_v[j, sl]
            sv = src_v[j, sl]
            tv = et_v[j, sl]
            vsi = plsc.load_gather(si_v, [dv])
            vsj = plsc.load_gather(sj_v, [sv])
            vsr = plsc.load_gather(srm_v, [tv, zero16])
            lg = vsi + vsj + vsr
            lg = jnp.where(lg >= 0, lg, 0.2 * lg)
            e_v[j, sl] = jnp.exp(lg - m_bound)
        pltpu.sync_copy(e_v.at[j], ssum_sh.at[dst_v.at[j]], add=True)
        return 0
    lax.fori_loop(0, JPT, per_chunk, 0)

    pltpu.sync_copy(e_v, e_h.at[pl.ds(base, JPT)])
    plsc.subcore_barrier()
    pltpu.sync_copy(ssum_sh.at[pl.ds(s * RPT, RPT)],
                    ssum_h.at[c, pl.ds(s * RPT, RPT)])


_SC_PARAMS = pltpu.CompilerParams(needs_layout_passes=False)

_sc_logits = pl.kernel(
    _sc_logits_body,
    compiler_params=_SC_PARAMS,
    out_type=(
        jax.ShapeDtypeStruct((EJ, 128), _f32),
        jax.ShapeDtypeStruct((NC, NP), _f32),
    ),
    mesh=_mesh,
    scratch_types=[
        pltpu.VMEM((JPT, 128), _i32),
        pltpu.VMEM((JPT, 128), _i32),
        pltpu.VMEM((JPT, 128), _i32),
        pltpu.VMEM((JPT, 128), _f32),
        pltpu.VMEM((NP,), _f32),
        pltpu.VMEM((NP,), _f32),
        pltpu.VMEM((8, 128), _f32),
        pltpu.VMEM((RPT,), _f32),
        pltpu.VMEM_SHARED((NP,), _f32),
    ],
)


STRIP = 8
NSTRIP = JPT // STRIP


def _make_sc_agg(with_pre):
    def body(*args):
        if with_pre:
            (src_h, dst_h, e_h, r_h, g_h, pre_h, agg_h,
             src_s, dst_s, e_s, pre_s, alpha_s, r_v, rows_a, rows_b,
             agg_sh, sema, semb) = args
        else:
            (src_h, dst_h, e_h, r_h, g_h, alpha_h, agg_h,
             src_s, dst_s, e_s, alpha_s, r_v, rows_a, rows_b,
             agg_sh, sema, semb) = args
        c = lax.axis_index("c")
        s = lax.axis_index("s")
        wid = c * NS + s
        base = wid * JPT

        pltpu.sync_copy(r_h, r_v)

        def zr(r, _):
            for f in range(8):
                rows_a[r, pl.ds(f * 16, 16)] = _zvec()
            return 0
        lax.fori_loop(0, 128, zr, 0)
        for k in range(RPT // 128):
            pltpu.sync_copy(rows_a, agg_sh.at[pl.ds(s * RPT + k * 128, 128)])
        plsc.subcore_barrier()

        bufs = (rows_a, rows_b)
        sems = (sema, semb)

        def per_strip(t, _):
            row0 = pl.multiple_of(base + t * STRIP, 8)
            pltpu.sync_copy(src_h.at[pl.ds(row0, STRIP)], src_s)
            pltpu.sync_copy(dst_h.at[pl.ds(row0, STRIP)], dst_s)
            pltpu.sync_copy(e_h.at[pl.ds(row0, STRIP)], e_s)
            if with_pre:
                pltpu.sync_copy(pre_h.at[pl.ds(row0, STRIP)], pre_s)

            cp = pltpu.async_copy(g_h.at[src_s.at[0]], rows_a, sema)
            for j in range(STRIP):
                buf = bufs[j % 2]
                if j + 1 < STRIP:
                    cp_next = pltpu.async_copy(
                        g_h.at[src_s.at[j + 1]], bufs[(j + 1) % 2],
                        sems[(j + 1) % 2])
                for i in range(8):
                    sl = pl.ds(i * 16, 16)
                    ev = e_s[j, sl]
                    dv = dst_s[j, sl]
                    hi = lax.shift_right_logical(dv, 7)
                    lo = jnp.bitwise_and(dv, 127)
                    rv = plsc.load_gather(r_v, [hi, lo])
                    av = ev * rv
                    if with_pre:
                        av = av * (1.0 - BETA) + pre_s[j, sl] * BETA
                    alpha_s[j, sl] = av
                cp.wait()

                def rbody(ri, _):
                    av = alpha_s[j, pl.ds(ri * 16, 16)]
                    for r0 in range(16):
                        sc = av[r0]
                        row = ri * 16 + r0
                        for f in range(8):
                            fl = pl.ds(f * 16, 16)
                            buf[row, fl] = buf[row, fl] * sc
                    return 0
                lax.fori_loop(0, 8, rbody, 0)
                pltpu.sync_copy(buf, agg_sh.at[dst_s.at[j]], add=True)
                if j + 1 < STRIP:
                    cp = cp_next
            if not with_pre:
                pltpu.sync_copy(alpha_s, alpha_h.at[pl.ds(row0, STRIP)])
            return 0
        lax.fori_loop(0, NSTRIP, per_strip, 0)

        plsc.subcore_barrier()
        pltpu.sync_copy(agg_sh.at[pl.ds(s * RPT, RPT)],
                        agg_h.at[c, pl.ds(s * RPT, RPT)])

    outs = [jax.ShapeDtypeStruct((NC, NP, 128), _f32)]
    if not with_pre:
        outs = [jax.ShapeDtypeStruct((EJ, 128), _f32)] + outs
    scratch = [
        pltpu.VMEM((STRIP, 128), _i32),
        pltpu.VMEM((STRIP, 128), _i32),
        pltpu.VMEM((STRIP, 128), _f32),
    ]
    if with_pre:
        scratch.append(pltpu.VMEM((STRIP, 128), _f32))
    scratch += [
        pltpu.VMEM((STRIP, 128), _f32),
        pltpu.VMEM((NP // 128, 128), _f32),
        pltpu.VMEM((128, 128), _f32),
        pltpu.VMEM((128, 128), _f32),
        pltpu.VMEM_SHARED((NP, 128), _f32),
        pltpu.SemaphoreType.DMA,
        pltpu.SemaphoreType.DMA,
    ]
    return pl.kernel(body, out_type=tuple(outs), mesh=_mesh,
                     compiler_params=_SC_PARAMS, scratch_types=scratch)


_sc_agg1 = _make_sc_agg(with_pre=False)
_sc_agg2 = _make_sc_agg(with_pre=True)


def _leaky(x, slope):
    return jnp.where(x >= 0, x, slope * x)


def _elu(x):
    return jnp.where(x > 0, x, jnp.exp(jnp.minimum(x, 0.0)) - 1.0)


BLK = 1000


def _tc1_body(f_ref, w1_ref, b1_ref, wl_ref, wres_ref, a2_ref, rel_ref,
              wr_ref, ar_ref, g_ref, sij_ref, xres_ref, srm_ref):
    h1 = _leaky(f_ref[...] @ w1_ref[...] + b1_ref[...], 0.01)
    g = h1 @ wl_ref[...]
    g_ref[...] = g
    sij_ref[...] = g @ a2_ref[...]
    xres_ref[...] = h1 @ wres_ref[...]
    srm_ref[...] = jnp.broadcast_to(
        (rel_ref[...] @ wr_ref[...]) @ ar_ref[...], (8, 128))


def _tc1(feature, W1, b1r, Wl, Wres, A2, relp, Wrp, AR):
    return pl.pallas_call(
        _tc1_body,
        grid=(N // BLK,),
        in_specs=[
            pl.BlockSpec((BLK, HID), lambda i: (i, 0)),
            pl.BlockSpec((HID, HID), lambda i: (0, 0)),
            pl.BlockSpec((1, HID), lambda i: (0, 0)),
            pl.BlockSpec((HID, HID), lambda i: (0, 0)),
            pl.BlockSpec((HID, HID), lambda i: (0, 0)),
            pl.BlockSpec((HID, 2), lambda i: (0, 0)),
            pl.BlockSpec((8, HID), lambda i: (0, 0)),
            pl.BlockSpec((HID, HID), lambda i: (0, 0)),
            pl.BlockSpec((HID, 1), lambda i: (0, 0)),
        ],
        out_specs=[
            pl.BlockSpec((BLK, HID), lambda i: (i, 0)),
            pl.BlockSpec((BLK, 2), lambda i: (i, 0)),
            pl.BlockSpec((BLK, HID), lambda i: (i, 0)),
            pl.BlockSpec((8, HID), lambda i: (0, 0)),
        ],
        out_shape=[
            jax.ShapeDtypeStruct((N, HID), _f32),
            jax.ShapeDtypeStruct((N, 2), _f32),
            jax.ShapeDtypeStruct((N, HID), _f32),
            jax.ShapeDtypeStruct((8, HID), _f32),
        ],
    )(feature, W1, b1r, Wl, Wres, A2, relp, Wrp, AR)


def _rsum_body(s_ref, r_ref):
    r_ref[...] = 1.0 / (s_ref[0] + s_ref[1] + 1e-16)


def _rsum(ssum):
    return pl.pallas_call(
        _rsum_body,
        out_shape=jax.ShapeDtypeStruct((NP // 128, 128), _f32),
    )(ssum.reshape(NC, NP // 128, 128))


def _tc2_body(agg_ref, xres_ref, bres_ref, wl_ref, wres_ref,
              a2_ref, g_ref, sij_ref, xres2_ref):
    x2 = _elu(agg_ref[0] + agg_ref[1] + xres_ref[...] + bres_ref[...])
    g = x2 @ wl_ref[...]
    g_ref[...] = g
    sij_ref[...] = g @ a2_ref[...]
    xres2_ref[...] = x2 @ wres_ref[...]


def _tc2(agg, xres1, bresr, Wl, Wres, A2):
    return pl.pallas_call(
        _tc2_body,
        grid=(N // BLK,),
        in_specs=[
            pl.BlockSpec((NC, BLK, HID), lambda i: (0, i, 0)),
            pl.BlockSpec((BLK, HID), lambda i: (i, 0)),
            pl.BlockSpec((1, HID), lambda i: (0, 0)),
            pl.BlockSpec((HID, HID), lambda i: (0, 0)),
            pl.BlockSpec((HID, HID), lambda i: (0, 0)),
            pl.BlockSpec((HID, 2), lambda i: (0, 0)),
        ],
        out_specs=[
            pl.BlockSpec((BLK, HID), lambda i: (i, 0)),
            pl.BlockSpec((BLK, 2), lambda i: (i, 0)),
            pl.BlockSpec((BLK, HID), lambda i: (i, 0)),
        ],
        out_shape=[
            jax.ShapeDtypeStruct((N, HID), _f32),
            jax.ShapeDtypeStruct((N, 2), _f32),
            jax.ShapeDtypeStruct((N, HID), _f32),
        ],
    )(agg, xres1, bresr, Wl, Wres, A2)


def _tc3_body(agg_ref, xres_ref, bres_ref, wo1_ref, bo1_ref,
              wo2_ref, bo2_ref, o_ref):
    x3 = _elu(agg_ref[0] + agg_ref[1] + xres_ref[...] + bres_ref[...])
    t = _leaky(x3 @ wo1_ref[...] + bo1_ref[...], 0.01)
    o_ref[...] = t @ wo2_ref[...] + bo2_ref[...]


def _tc3(agg, xres2, bresr, Wo1, bo1r, Wo2p, bo2r):
    return pl.pallas_call(
        _tc3_body,
        grid=(N // BLK,),
        in_specs=[
            pl.BlockSpec((NC, BLK, HID), lambda i: (0, i, 0)),
            pl.BlockSpec((BLK, HID), lambda i: (i, 0)),
            pl.BlockSpec((1, HID), lambda i: (0, 0)),
            pl.BlockSpec((HID, 64), lambda i: (0, 0)),
            pl.BlockSpec((1, 64), lambda i: (0, 0)),
            pl.BlockSpec((64, HID), lambda i: (0, 0)),
            pl.BlockSpec((1, HID), lambda i: (0, 0)),
        ],
        out_specs=pl.BlockSpec((BLK, HID), lambda i: (i, 0)),
        out_shape=jax.ShapeDtypeStruct((N, HID), _f32),
    )(agg, xres2, bresr, Wo1, bo1r, Wo2p, bo2r)


def kernel(feature, edge_index, edge_type, W1, b1, Wl, Wr, a, Wres, bres,
           rel_emb, Wo1, bo1, Wo2, bo2):
    src = edge_index[0].astype(_i32)
    dst = edge_index[1].astype(_i32)
    et = edge_type.astype(_i32)

    pad = EP - E
    pad_dst = N + (jnp.arange(pad, dtype=_i32) % (NP - N))
    pad_src = N + (jnp.arange(pad, dtype=_i32) % (NG - N))
    src2 = jnp.concatenate([src, pad_src]).reshape(EJ, 128)
    dst2 = jnp.concatenate([dst, pad_dst]).reshape(EJ, 128)
    et2 = jnp.concatenate([et, jnp.zeros((pad,), _i32)]).reshape(EJ, 128)

    b1r = b1.reshape(1, HID)
    bresr = bres.reshape(1, HID)
    bo1r = bo1.reshape(1, 64)
    A2 = jnp.concatenate([a[0:HID], a[HID:2 * HID]], axis=1)
    AR = a[2 * HID:3 * HID]
    relp = jnp.zeros((8, HID), _f32).at[:4, :100].set(rel_emb)
    Wrp = jnp.zeros((HID, HID), _f32).at[:100].set(Wr)
    Wo2p = jnp.zeros((64, HID), _f32).at[:, :2].set(Wo2)
    bo2r = jnp.zeros((1, HID), _f32).at[0, :2].set(bo2)

    g1, sij1, xres1, srm = _tc1(feature, W1, b1r, Wl, Wres, A2, relp, Wrp, AR)
    si1 = jnp.pad(sij1[:, 0], (0, NP - N))
    sj1 = jnp.pad(sij1[:, 1], (0, NP - N))
    g1p = jnp.pad(g1, ((0, NG - N), (0, 0)))

    e1, ssum1 = _sc_logits(src2, dst2, et2, si1, sj1, srm)
    alpha1, agg1 = _sc_agg1(src2, dst2, e1, _rsum(ssum1), g1p)

    g2, sij2, xres2 = _tc2(agg1, xres1, bresr, Wl, Wres, A2)
    si2 = jnp.pad(sij2[:, 0], (0, NP - N))
    sj2 = jnp.pad(sij2[:, 1], (0, NP - N))
    g2p = jnp.pad(g2, ((0, NG - N), (0, 0)))

    e2, ssum2 = _sc_logits(src2, dst2, et2, si2, sj2, srm)
    (agg2,) = _sc_agg2(src2, dst2, e2, _rsum(ssum2), g2p, alpha1)

    out = _tc3(agg2, xres2, bresr, Wo1, bo1r, Wo2p, bo2r)
    return out[:, :2]


# speedup vs baseline: 2.5430x; 1.0326x over previous
import functools

import jax
import jax.numpy as jnp
from jax import lax
from jax.experimental import pallas as pl
from jax.experimental.pallas import tpu as pltpu
from jax.experimental.pallas import tpu_sc as plsc

N = 10000
E = 320000
HID = 128
BETA = 0.05
NC, NS = 2, 16
NW = NC * NS
JPT = 80
EPT = JPT * 128
EP = EPT * NW
EJ = EP // 128
NP = 10240
NG = 10016
RPT = NP // NS

_f32 = jnp.float32
_i32 = jnp.int32

_mesh = plsc.VectorSubcoreMesh(core_axis_name="c", subcore_axis_name="s")


def _zvec():
    return jnp.zeros((16,), _f32)


def _sc_logits_body(src_h, dst_h, et_h, si_h, sj_h, srm_h, e_h, ssum_h,
                    src_v, dst_v, et_v, e_v, si_v, sj_v, srm_v, zbuf,
                    ssum_sh, sem):
    c = lax.axis_index("c")
    s = lax.axis_index("s")
    wid = c * NS + s
    base = wid * JPT

    def zb(k, _):
        zbuf[pl.ds(k * 16, 16)] = _zvec()
        return 0
    lax.fori_loop(0, RPT // 16, zb, 0)
    pltpu.sync_copy(zbuf, ssum_sh.at[pl.ds(s * RPT, RPT)])

    pltpu.sync_copy(src_h.at[pl.ds(base, JPT)], src_v)
    pltpu.sync_copy(dst_h.at[pl.ds(base, JPT)], dst_v)
    pltpu.sync_copy(et_h.at[pl.ds(base, JPT)], et_v)
    pltpu.sync_copy(si_h, si_v)
    pltpu.sync_copy(sj_h, sj_v)
    pltpu.sync_copy(srm_h, srm_v)
    plsc.subcore_barrier()

    def mxi(i, cm):
        return jnp.maximum(cm, si_v[pl.ds(i * 16, 16)])

    def mxj(i, cm):
        return jnp.maximum(cm, sj_v[pl.ds(i * 16, 16)])

    def _lanes_max(v):
        m = v[0]
        for i in range(1, 16):
            m = jnp.maximum(m, v[i])
        return m

    neg = jnp.full((16,), -1e30, _f32)
    msi = _lanes_max(lax.fori_loop(0, NP // 16, mxi, neg))
    msj = _lanes_max(lax.fori_loop(0, NP // 16, mxj, neg))
    sr0 = srm_v[0, pl.ds(0, 16)]
    sr1 = srm_v[1, pl.ds(0, 16)]
    sr2 = srm_v[2, pl.ds(0, 16)]
    sr3 = srm_v[3, pl.ds(0, 16)]
    msr = jnp.maximum(jnp.maximum(sr0, sr1), jnp.maximum(sr2, sr3))[0]
    mb = msi + msj + msr
    m_bound = jnp.where(mb >= 0, mb, 0.2 * mb)

    zero16 = jnp.zeros((16,), _i32)

    def _drain_one():
        pltpu.make_async_copy(e_v.at[0], ssum_sh.at[dst_v.at[0]], sem).wait()

    def per_chunk(j, _):
        for i in range(8):
            sl = pl.ds(i * 16, 16)
            dv = dst_v[j, sl]
            sv = src_v[j, sl]
            tv = et_v[j, sl]
            vsi = plsc.load_gather(si_v, [dv])
            vsj = plsc.load_gather(sj_v, [sv])
            vsr = plsc.load_gather(srm_v, [tv, zero16])
            lg = vsi + vsj + vsr
            lg = jnp.where(lg >= 0, lg, 0.2 * lg)
            e_v[j, sl] = jnp.exp(lg - m_bound)
        pltpu.async_copy(e_v.at[j], ssum_sh.at[dst_v.at[j]], sem, add=True)

        @pl.when(j >= 8)
        def _():
            _drain_one()
        return 0
    lax.fori_loop(0, JPT, per_chunk, 0)

    def drain(j, _):
        _drain_one()
        return 0
    lax.fori_loop(0, 8, drain, 0)

    pltpu.sync_copy(e_v, e_h.at[pl.ds(base, JPT)])
    plsc.subcore_barrier()
    pltpu.sync_copy(ssum_sh.at[pl.ds(s * RPT, RPT)],
                    ssum_h.at[c, pl.ds(s * RPT, RPT)])


_SC_PARAMS = pltpu.CompilerParams(needs_layout_passes=False)

_sc_logits = pl.kernel(
    _sc_logits_body,
    compiler_params=_SC_PARAMS,
    out_type=(
        jax.ShapeDtypeStruct((EJ, 128), _f32),
        jax.ShapeDtypeStruct((NC, NP), _f32),
    ),
    mesh=_mesh,
    scratch_types=[
        pltpu.VMEM((JPT, 128), _i32),
        pltpu.VMEM((JPT, 128), _i32),
        pltpu.VMEM((JPT, 128), _i32),
        pltpu.VMEM((JPT, 128), _f32),
        pltpu.VMEM((NP,), _f32),
        pltpu.VMEM((NP,), _f32),
        pltpu.VMEM((8, 128), _f32),
        pltpu.VMEM((RPT,), _f32),
        pltpu.VMEM_SHARED((NP,), _f32),
        pltpu.SemaphoreType.DMA,
    ],
)


STRIP = 8
NSTRIP = JPT // STRIP


def _make_sc_agg(with_pre):
    def body(*args):
        if with_pre:
            (src_h, dst_h, e_h, r_h, g_h, pre_h, agg_h,
             src_s, dst_s, e_s, pre_s, alpha_s, r_v, rows_a, rows_b,
             agg_sh, sema, semb, sca, scb) = args
        else:
            (src_h, dst_h, e_h, r_h, g_h, alpha_h, agg_h,
             src_s, dst_s, e_s, alpha_s, r_v, rows_a, rows_b,
             agg_sh, sema, semb, sca, scb) = args
        c = lax.axis_index("c")
        s = lax.axis_index("s")
        wid = c * NS + s
        base = wid * JPT

        pltpu.sync_copy(r_h, r_v)

        def zr(r, _):
            for f in range(8):
                rows_a[r, pl.ds(f * 16, 16)] = _zvec()
            return 0
        lax.fori_loop(0, 128, zr, 0)
        for k in range(RPT // 128):
            pltpu.sync_copy(rows_a, agg_sh.at[pl.ds(s * RPT + k * 128, 128)])
        plsc.subcore_barrier()

        bufs = (rows_a, rows_b)
        gsems = (sema, semb)
        ssems = (sca, scb)

        def _wait_scatter(b):
            pltpu.make_async_copy(bufs[b], agg_sh.at[dst_s.at[0]],
                                  ssems[b]).wait()

        def strip_body(t, first):
            row0 = pl.multiple_of(base + t * STRIP, 8)
            pltpu.sync_copy(src_h.at[pl.ds(row0, STRIP)], src_s)
            pltpu.sync_copy(dst_h.at[pl.ds(row0, STRIP)], dst_s)
            pltpu.sync_copy(e_h.at[pl.ds(row0, STRIP)], e_s)
            if with_pre:
                pltpu.sync_copy(pre_h.at[pl.ds(row0, STRIP)], pre_s)

            if not first:
                _wait_scatter(0)
            cp = [pltpu.async_copy(g_h.at[src_s.at[0]], rows_a, sema), None]
            for j in range(STRIP):
                b = j % 2
                buf = bufs[b]
                if j + 1 < STRIP:
                    ob = 1 - b
                    if not (first and j == 0):
                        _wait_scatter(ob)
                    cp[ob] = pltpu.async_copy(
                        g_h.at[src_s.at[j + 1]], bufs[ob], gsems[ob])
                for i in range(8):
                    sl = pl.ds(i * 16, 16)
                    ev = e_s[j, sl]
                    dv = dst_s[j, sl]
                    hi = lax.shift_right_logical(dv, 7)
                    lo = jnp.bitwise_and(dv, 127)
                    rv = plsc.load_gather(r_v, [hi, lo])
                    av = ev * rv
                    if with_pre:
                        av = av * (1.0 - BETA) + pre_s[j, sl] * BETA
                    alpha_s[j, sl] = av
                cp[b].wait()

                def rbody(ri, _):
                    av = alpha_s[j, pl.ds(ri * 16, 16)]
                    for r0 in range(16):
                        sc = av[r0]
                        row = ri * 16 + r0
                        for f in range(8):
                            fl = pl.ds(f * 16, 16)
                            buf[row, fl] = buf[row, fl] * sc
                    return 0
                lax.fori_loop(0, 8, rbody, 0)
                pltpu.async_copy(buf, agg_sh.at[dst_s.at[j]], ssems[b],
                                 add=True)
            if not with_pre:
                pltpu.sync_copy(alpha_s, alpha_h.at[pl.ds(row0, STRIP)])

        strip_body(0, True)

        def per_strip(t, _):
            strip_body(t, False)
            return 0
        lax.fori_loop(1, NSTRIP, per_strip, 0)

        _wait_scatter(0)
        _wait_scatter(1)
        plsc.subcore_barrier()
        pltpu.sync_copy(agg_sh.at[pl.ds(s * RPT, RPT)],
                        agg_h.at[c, pl.ds(s * RPT, RPT)])

    outs = [jax.ShapeDtypeStruct((NC, NP, 128), _f32)]
    if not with_pre:
        outs = [jax.ShapeDtypeStruct((EJ, 128), _f32)] + outs
    scratch = [
        pltpu.VMEM((STRIP, 128), _i32),
        pltpu.VMEM((STRIP, 128), _i32),
        pltpu.VMEM((STRIP, 128), _f32),
    ]
    if with_pre:
        scratch.append(pltpu.VMEM((STRIP, 128), _f32))
    scratch += [
        pltpu.VMEM((STRIP, 128), _f32),
        pltpu.VMEM((NP // 128, 128), _f32),
        pltpu.VMEM((128, 128), _f32),
        pltpu.VMEM((128, 128), _f32),
        pltpu.VMEM_SHARED((NP, 128), _f32),
        pltpu.SemaphoreType.DMA,
        pltpu.SemaphoreType.DMA,
        pltpu.SemaphoreType.DMA,
        pltpu.SemaphoreType.DMA,
    ]
    return pl.kernel(body, out_type=tuple(outs), mesh=_mesh,
                     compiler_params=_SC_PARAMS, scratch_types=scratch)


_sc_agg1 = _make_sc_agg(with_pre=False)
_sc_agg2 = _make_sc_agg(with_pre=True)


def _leaky(x, slope):
    return jnp.where(x >= 0, x, slope * x)


def _elu(x):
    return jnp.where(x > 0, x, jnp.exp(jnp.minimum(x, 0.0)) - 1.0)


BLK = 1000


def _tc1_body(f_ref, w1_ref, b1_ref, wl_ref, wres_ref, a2_ref, rel_ref,
              wr_ref, ar_ref, g_ref, sij_ref, xres_ref, srm_ref):
    h1 = _leaky(f_ref[...] @ w1_ref[...] + b1_ref[...], 0.01)
    g = h1 @ wl_ref[...]
    g_ref[...] = g
    sij_ref[...] = g @ a2_ref[...]
    xres_ref[...] = h1 @ wres_ref[...]
    srm_ref[...] = jnp.broadcast_to(
        (rel_ref[...] @ wr_ref[...]) @ ar_ref[...], (8, 128))


def _tc1(feature, W1, b1r, Wl, Wres, A2, relp, Wrp, AR):
    return pl.pallas_call(
        _tc1_body,
        grid=(N // BLK,),
        in_specs=[
            pl.BlockSpec((BLK, HID), lambda i: (i, 0)),
            pl.BlockSpec((HID, HID), lambda i: (0, 0)),
            pl.BlockSpec((1, HID), lambda i: (0, 0)),
            pl.BlockSpec((HID, HID), lambda i: (0, 0)),
            pl.BlockSpec((HID, HID), lambda i: (0, 0)),
            pl.BlockSpec((HID, 2), lambda i: (0, 0)),
            pl.BlockSpec((8, HID), lambda i: (0, 0)),
            pl.BlockSpec((HID, HID), lambda i: (0, 0)),
            pl.BlockSpec((HID, 1), lambda i: (0, 0)),
        ],
        out_specs=[
            pl.BlockSpec((BLK, HID), lambda i: (i, 0)),
            pl.BlockSpec((BLK, 2), lambda i: (i, 0)),
            pl.BlockSpec((BLK, HID), lambda i: (i, 0)),
            pl.BlockSpec((8, HID), lambda i: (0, 0)),
        ],
        out_shape=[
            jax.ShapeDtypeStruct((N, HID), _f32),
            jax.ShapeDtypeStruct((N, 2), _f32),
            jax.ShapeDtypeStruct((N, HID), _f32),
            jax.ShapeDtypeStruct((8, HID), _f32),
        ],
    )(feature, W1, b1r, Wl, Wres, A2, relp, Wrp, AR)


def _rsum_body(s_ref, r_ref):
    r_ref[...] = 1.0 / (s_ref[0] + s_ref[1] + 1e-16)


def _rsum(ssum):
    return pl.pallas_call(
        _rsum_body,
        out_shape=jax.ShapeDtypeStruct((NP // 128, 128), _f32),
    )(ssum.reshape(NC, NP // 128, 128))


def _tc2_body(agg_ref, xres_ref, bres_ref, wl_ref, wres_ref,
              a2_ref, g_ref, sij_ref, xres2_ref):
    x2 = _elu(agg_ref[0] + agg_ref[1] + xres_ref[...] + bres_ref[...])
    g = x2 @ wl_ref[...]
    g_ref[...] = g
    sij_ref[...] = g @ a2_ref[...]
    xres2_ref[...] = x2 @ wres_ref[...]


def _tc2(agg, xres1, bresr, Wl, Wres, A2):
    return pl.pallas_call(
        _tc2_body,
        grid=(N // BLK,),
        in_specs=[
            pl.BlockSpec((NC, BLK, HID), lambda i: (0, i, 0)),
            pl.BlockSpec((BLK, HID), lambda i: (i, 0)),
            pl.BlockSpec((1, HID), lambda i: (0, 0)),
            pl.BlockSpec((HID, HID), lambda i: (0, 0)),
            pl.BlockSpec((HID, HID), lambda i: (0, 0)),
            pl.BlockSpec((HID, 2), lambda i: (0, 0)),
        ],
        out_specs=[
            pl.BlockSpec((BLK, HID), lambda i: (i, 0)),
            pl.BlockSpec((BLK, 2), lambda i: (i, 0)),
            pl.BlockSpec((BLK, HID), lambda i: (i, 0)),
        ],
        out_shape=[
            jax.ShapeDtypeStruct((N, HID), _f32),
            jax.ShapeDtypeStruct((N, 2), _f32),
            jax.ShapeDtypeStruct((N, HID), _f32),
        ],
    )(agg, xres1, bresr, Wl, Wres, A2)


def _tc3_body(agg_ref, xres_ref, bres_ref, wo1_ref, bo1_ref,
              wo2_ref, bo2_ref, o_ref):
    x3 = _elu(agg_ref[0] + agg_ref[1] + xres_ref[...] + bres_ref[...])
    t = _leaky(x3 @ wo1_ref[...] + bo1_ref[...], 0.01)
    o_ref[...] = t @ wo2_ref[...] + bo2_ref[...]


def _tc3(agg, xres2, bresr, Wo1, bo1r, Wo2p, bo2r):
    return pl.pallas_call(
        _tc3_body,
        grid=(N // BLK,),
        in_specs=[
            pl.BlockSpec((NC, BLK, HID), lambda i: (0, i, 0)),
            pl.BlockSpec((BLK, HID), lambda i: (i, 0)),
            pl.BlockSpec((1, HID), lambda i: (0, 0)),
            pl.BlockSpec((HID, 64), lambda i: (0, 0)),
            pl.BlockSpec((1, 64), lambda i: (0, 0)),
            pl.BlockSpec((64, HID), lambda i: (0, 0)),
            pl.BlockSpec((1, HID), lambda i: (0, 0)),
        ],
        out_specs=pl.BlockSpec((BLK, HID), lambda i: (i, 0)),
        out_shape=jax.ShapeDtypeStruct((N, HID), _f32),
    )(agg, xres2, bresr, Wo1, bo1r, Wo2p, bo2r)


def kernel(feature, edge_index, edge_type, W1, b1, Wl, Wr, a, Wres, bres,
           rel_emb, Wo1, bo1, Wo2, bo2):
    src = edge_index[0].astype(_i32)
    dst = edge_index[1].astype(_i32)
    et = edge_type.astype(_i32)

    pad = EP - E
    pad_dst = N + (jnp.arange(pad, dtype=_i32) % (NP - N))
    pad_src = N + (jnp.arange(pad, dtype=_i32) % (NG - N))
    src2 = jnp.concatenate([src, pad_src]).reshape(EJ, 128)
    dst2 = jnp.concatenate([dst, pad_dst]).reshape(EJ, 128)
    et2 = jnp.concatenate([et, jnp.zeros((pad,), _i32)]).reshape(EJ, 128)

    b1r = b1.reshape(1, HID)
    bresr = bres.reshape(1, HID)
    bo1r = bo1.reshape(1, 64)
    A2 = jnp.concatenate([a[0:HID], a[HID:2 * HID]], axis=1)
    AR = a[2 * HID:3 * HID]
    relp = jnp.zeros((8, HID), _f32).at[:4, :100].set(rel_emb)
    Wrp = jnp.zeros((HID, HID), _f32).at[:100].set(Wr)
    Wo2p = jnp.zeros((64, HID), _f32).at[:, :2].set(Wo2)
    bo2r = jnp.zeros((1, HID), _f32).at[0, :2].set(bo2)

    g1, sij1, xres1, srm = _tc1(feature, W1, b1r, Wl, Wres, A2, relp, Wrp, AR)
    si1 = jnp.pad(sij1[:, 0], (0, NP - N))
    sj1 = jnp.pad(sij1[:, 1], (0, NP - N))
    g1p = jnp.pad(g1, ((0, NG - N), (0, 0)))

    e1, ssum1 = _sc_logits(src2, dst2, et2, si1, sj1, srm)
    alpha1, agg1 = _sc_agg1(src2, dst2, e1, _rsum(ssum1), g1p)

    g2, sij2, xres2 = _tc2(agg1, xres1, bresr, Wl, Wres, A2)
    si2 = jnp.pad(sij2[:, 0], (0, NP - N))
    sj2 = jnp.pad(sij2[:, 1], (0, NP - N))
    g2p = jnp.pad(g2, ((0, NG - N), (0, 0)))

    e2, ssum2 = _sc_logits(src2, dst2, et2, si2, sj2, srm)
    (agg2,) = _sc_agg2(src2, dst2, e2, _rsum(ssum2), g2p, alpha1)

    out = _tc3(agg2, xres2, bresr, Wo1, bo1r, Wo2p, bo2r)
    return out[:, :2]
